# baseline (speedup 1.0000x reference)
import functools

import jax
import jax.numpy as jnp
from jax import lax
from jax.experimental import pallas as pl
from jax.experimental.pallas import tpu as pltpu
from jax.experimental.pallas import tpu_sc as plsc

N = 10000
E = 32000
T = 5
F = 404
H = 128
L = 3
Q = 32000

NP = 10240
FP = 512
EP = 32768
QP = 32768
CH = 128
ZB = 64
RB = 512
QB = 512
NTILE = 16
NCORE = 2

f32 = jnp.float32


def _r16(v):
    return v.astype(jnp.bfloat16).astype(f32)


def _inproj_body(x_ref, w_ref, b_ref, o_ref):
    o_ref[...] = _r16(jnp.dot(_r16(x_ref[...]), _r16(w_ref[...]),
                              preferred_element_type=f32,
                              precision=lax.Precision.HIGHEST) + b_ref[...])


_inproj = pl.pallas_call(
    _inproj_body,
    grid=(NP // RB,),
    in_specs=[
        pl.BlockSpec((RB, FP), lambda i: (i, 0)),
        pl.BlockSpec((FP, H), lambda i: (0, 0)),
        pl.BlockSpec((1, H), lambda i: (0, 0)),
    ],
    out_specs=pl.BlockSpec((RB, H), lambda i: (i, 0)),
    out_shape=jax.ShapeDtypeStruct((NP, H), f32),
)


def _cntprep_body(c_ref, o_ref):
    cnt = c_ref[0, 0, :, 0:1] + c_ref[0, 1, :, 0:1]
    recip = 1.0 / jnp.clip(cnt, 1.0, None)
    o_ref[...] = jnp.broadcast_to(recip, (RB, 8))[None]


_cntprep = pl.pallas_call(
    _cntprep_body,
    grid=(T, NP // RB),
    in_specs=[pl.BlockSpec((1, NCORE, RB, H), lambda t, i: (t, 0, i, 0))],
    out_specs=pl.BlockSpec((1, RB, 8), lambda t, i: (t, i, 0)),
    out_shape=jax.ShapeDtypeStruct((T, NP, 8), f32),
)


def _layer_body(h_ref, s_ref, r_ref, wself_ref, wmsg_ref, b_ref, o_ref):
    acc = jnp.dot(h_ref[...], _r16(wself_ref[...]),
                  preferred_element_type=f32, precision=lax.Precision.HIGHEST)
    for t in range(T):
        st = s_ref[t, 0] + s_ref[t, 1]
        recip = r_ref[t, :, 0:1]
        acc = acc + jnp.dot(st, _r16(wmsg_ref[t]),
                            preferred_element_type=f32,
                            precision=lax.Precision.HIGHEST) * recip
    o_ref[...] = _r16(jnp.maximum(acc + b_ref[...], 0.0))


_layer = pl.pallas_call(
    _layer_body,
    grid=(NP // RB,),
    in_specs=[
        pl.BlockSpec((RB, H), lambda i: (i, 0)),
        pl.BlockSpec((T, NCORE, RB, H), lambda i: (0, 0, i, 0)),
        pl.BlockSpec((T, RB, 8), lambda i: (0, i, 0)),
        pl.BlockSpec((H, H), lambda i: (0, 0)),
        pl.BlockSpec((T, H, H), lambda i: (0, 0, 0)),
        pl.BlockSpec((1, H), lambda i: (0, 0)),
    ],
    out_specs=pl.BlockSpec((RB, H), lambda i: (i, 0)),
    out_shape=jax.ShapeDtypeStruct((NP, H), f32),
)


def _layer_uv_body(h_ref, s_ref, r_ref, wself_ref, wmsg_ref, b_ref,
                   wa_ref, wb_ref, u_ref, v_ref):
    acc = jnp.dot(h_ref[...], _r16(wself_ref[...]),
                  preferred_element_type=f32, precision=lax.Precision.HIGHEST)
    for t in range(T):
        st = s_ref[t, 0] + s_ref[t, 1]
        recip = r_ref[t, :, 0:1]
        acc = acc + jnp.dot(st, _r16(wmsg_ref[t]),
                            preferred_element_type=f32,
                            precision=lax.Precision.HIGHEST) * recip
    out = _r16(jnp.maximum(acc + b_ref[...], 0.0))
    u_ref[...] = jnp.dot(out, _r16(wa_ref[...]),
                         preferred_element_type=f32,
                         precision=lax.Precision.HIGHEST)
    v_ref[...] = jnp.dot(out, _r16(wb_ref[...]),
                         preferred_element_type=f32,
                         precision=lax.Precision.HIGHEST)


_layer_uv = pl.pallas_call(
    _layer_uv_body,
    grid=(NP // RB,),
    in_specs=[
        pl.BlockSpec((RB, H), lambda i: (i, 0)),
        pl.BlockSpec((T, NCORE, RB, H), lambda i: (0, 0, i, 0)),
        pl.BlockSpec((T, RB, 8), lambda i: (0, i, 0)),
        pl.BlockSpec((H, H), lambda i: (0, 0)),
        pl.BlockSpec((T, H, H), lambda i: (0, 0, 0)),
        pl.BlockSpec((1, H), lambda i: (0, 0)),
        pl.BlockSpec((H, H), lambda i: (0, 0)),
        pl.BlockSpec((H, H), lambda i: (0, 0)),
    ],
    out_specs=[
        pl.BlockSpec((RB, H), lambda i: (i, 0)),
        pl.BlockSpec((RB, H), lambda i: (i, 0)),
    ],
    out_shape=[
        jax.ShapeDtypeStruct((NP, H), f32),
        jax.ShapeDtypeStruct((NP, H), f32),
    ],
)


_QBLOCKS_PER_T = QP // QB


def _dec_body(gu_ref, gv_ref, temb_ref, w1c_ref, b1_ref, w2_ref, b2_ref, o_ref):
    t = pl.program_id(0) // _QBLOCKS_PER_T
    et = temb_ref[pl.ds(t, 1), :]
    c = jnp.dot(_r16(et), _r16(w1c_ref[...]),
                preferred_element_type=f32,
                precision=lax.Precision.HIGHEST) + b1_ref[...]
    z = _r16(jnp.maximum(gu_ref[0] + gv_ref[0] + c, 0.0))
    o_ref[...] = jnp.sum(z * _r16(w2_ref[...]), axis=1) + b2_ref[0]


_dec = pl.pallas_call(
    _dec_body,
    grid=(T * QP // QB,),
    in_specs=[
        pl.BlockSpec((1, QB, H), lambda i: (0, i, 0)),
        pl.BlockSpec((1, QB, H), lambda i: (1, i, 0)),
        pl.BlockSpec((T, H), lambda i: (0, 0)),
        pl.BlockSpec((H, H), lambda i: (0, 0)),
        pl.BlockSpec((1, H), lambda i: (0, 0)),
        pl.BlockSpec((1, H), lambda i: (0, 0)),
        pl.BlockSpec(memory_space=pltpu.SMEM),
    ],
    out_specs=pl.BlockSpec((QB,), lambda i: (i,)),
    out_shape=jax.ShapeDtypeStruct((T * QP,), f32),
)


_sc_mesh = plsc.VectorSubcoreMesh(core_axis_name="c", subcore_axis_name="s")

_ETILE = EP // (NCORE * NTILE)
_NCH = _ETILE // CH
_STRIPE = NP // NTILE


@functools.partial(
    pl.kernel,
    mesh=_sc_mesh,
    out_type=jax.ShapeDtypeStruct((T * NCORE * NP, H), f32),
    scratch_types=[
        pltpu.VMEM((_NCH, CH), jnp.int32),
        pltpu.VMEM((_NCH, CH), jnp.int32),
        pltpu.VMEM((2, CH, H), f32),
        pltpu.VMEM((ZB, H), f32),
        pltpu.VMEM_SHARED((NP, H), f32),
        pltpu.SemaphoreType.DMA,
        pltpu.SemaphoreType.DMA,
    ],
)
def _seg(h_hbm, src_hbm, dst_hbm, zeros_hbm, out_hbm,
         src_v, dst_v, rows_v, zero_v, acc_sh, sem0, sem1):
    c = lax.axis_index("c")
    s = lax.axis_index("s")
    pltpu.sync_copy(zeros_hbm, zero_v)
    for t in range(T):
        r = (t * NCORE + c) * NTILE + s
        pltpu.sync_copy(src_hbm.at[r], src_v)
        pltpu.sync_copy(dst_hbm.at[r], dst_v)
        for z in range(_STRIPE // ZB):
            pltpu.sync_copy(zero_v, acc_sh.at[pl.ds(s * _STRIPE + z * ZB, ZB)])
        plsc.subcore_barrier()
        sems = [sem0, sem1]
        gh = [None] * _NCH
        gh[0] = pltpu.async_copy(h_hbm.at[src_v.at[0]], rows_v.at[0], sem0)
        for j in range(_NCH):
            b = j % 2
            gh[j].wait()
            if j + 1 < _NCH:
                gh[j + 1] = pltpu.async_copy(h_hbm.at[src_v.at[j + 1]],
                                             rows_v.at[1 - b], sems[1 - b])
            pltpu.sync_copy(rows_v.at[b], acc_sh.at[dst_v.at[j]], add=True)
        plsc.subcore_barrier()
        out_row = (t * NCORE + c) * NP + s * _STRIPE
        pltpu.sync_copy(acc_sh.at[pl.ds(s * _STRIPE, _STRIPE)],
                        out_hbm.at[pl.ds(out_row, _STRIPE)])


@functools.partial(
    pl.kernel,
    mesh=_sc_mesh,
    out_type=jax.ShapeDtypeStruct((2 * T * NCORE * NP, H), f32),
    scratch_types=[
        pltpu.VMEM((_NCH, CH), jnp.int32),
        pltpu.VMEM((_NCH, CH), jnp.int32),
        pltpu.VMEM((2, CH, H), f32),
        pltpu.VMEM((ZB, H), f32),
        pltpu.VMEM_SHARED((NP, H), f32),
        pltpu.SemaphoreType.DMA,
        pltpu.SemaphoreType.DMA,
    ],
)
def _seg0(h_hbm, src_hbm, dst_hbm, zeros_hbm, ones_hbm, out_hbm,
          src_v, dst_v, rows_v, zero_v, acc_sh, sem0, sem1):
    c = lax.axis_index("c")
    s = lax.axis_index("s")
    sems = [sem0, sem1]
    pltpu.sync_copy(zeros_hbm, zero_v)
    for t in range(T):
        r = (t * NCORE + c) * NTILE + s
        pltpu.sync_copy(src_hbm.at[r], src_v)
        pltpu.sync_copy(dst_hbm.at[r], dst_v)
        for z in range(_STRIPE // ZB):
            pltpu.sync_copy(zero_v, acc_sh.at[pl.ds(s * _STRIPE + z * ZB, ZB)])
        plsc.subcore_barrier()
        gh = [None] * _NCH
        gh[0] = pltpu.async_copy(h_hbm.at[src_v.at[0]], rows_v.at[0], sem0)
        for j in range(_NCH):
            b = j % 2
            gh[j].wait()
            if j + 1 < _NCH:
                gh[j + 1] = pltpu.async_copy(h_hbm.at[src_v.at[j + 1]],
                                             rows_v.at[1 - b], sems[1 - b])
            pltpu.sync_copy(rows_v.at[b], acc_sh.at[dst_v.at[j]], add=True)
        plsc.subcore_barrier()
        out_row = (t * NCORE + c) * NP + s * _STRIPE
        pltpu.sync_copy(acc_sh.at[pl.ds(s * _STRIPE, _STRIPE)],
                        out_hbm.at[pl.ds(out_row, _STRIPE)])
    pltpu.sync_copy(ones_hbm, rows_v.at[0])
    for t in range(T):
        r = (t * NCORE + c) * NTILE + s
        pltpu.sync_copy(dst_hbm.at[r], dst_v)
        for z in range(_STRIPE // ZB):
            pltpu.sync_copy(zero_v, acc_sh.at[pl.ds(s * _STRIPE + z * ZB, ZB)])
        plsc.subcore_barrier()
        sh = [pltpu.async_copy(rows_v.at[0], acc_sh.at[dst_v.at[j]], sem0,
                               add=True)
              for j in range(_NCH)]
        for x in sh:
            x.wait()
        plsc.subcore_barrier()
        out_row = ((T + t) * NCORE + c) * NP + s * _STRIPE
        pltpu.sync_copy(acc_sh.at[pl.ds(s * _STRIPE, _STRIPE)],
                        out_hbm.at[pl.ds(out_row, _STRIPE)])


_PAIRS_PER_TILE = T * QP // (NCORE * NTILE)
_PCH = 64
_QCH = _PAIRS_PER_TILE // _PCH
_PD = 6
_PA = 3


@functools.partial(
    pl.kernel,
    mesh=_sc_mesh,
    out_type=jax.ShapeDtypeStruct((2 * T * QP, H), f32),
    scratch_types=[
        pltpu.VMEM((_QCH, _PCH), jnp.int32),
        pltpu.VMEM((_QCH, _PCH), jnp.int32),
        pltpu.VMEM((_PD, _PCH, H), f32),
        pltpu.VMEM((_PD, _PCH, H), f32),
        pltpu.SemaphoreType.DMA,
        pltpu.SemaphoreType.DMA,
        pltpu.SemaphoreType.DMA,
        pltpu.SemaphoreType.DMA,
        pltpu.SemaphoreType.DMA,
        pltpu.SemaphoreType.DMA,
        pltpu.SemaphoreType.DMA,
        pltpu.SemaphoreType.DMA,
        pltpu.SemaphoreType.DMA,
        pltpu.SemaphoreType.DMA,
        pltpu.SemaphoreType.DMA,
        pltpu.SemaphoreType.DMA,
        pltpu.SemaphoreType.DMA,
    ],
)
def _pair(u_hbm, v_hbm, qs_hbm, qt_hbm, out_hbm,
          qs_v, qt_v, bu_v, bv_v, isem,
          gsem0, gsem1, gsem2, gsem3, gsem4, gsem5,
          wsem0, wsem1, wsem2, wsem3, wsem4, wsem5):
    c = lax.axis_index("c")
    s = lax.axis_index("s")
    w = c * NTILE + s
    gsems = [gsem0, gsem1, gsem2, gsem3, gsem4, gsem5]
    wsems = [wsem0, wsem1, wsem2, wsem3, wsem4, wsem5]
    i1 = pltpu.async_copy(qs_hbm.at[w], qs_v, isem)
    i2 = pltpu.async_copy(qt_hbm.at[w], qt_v, isem)
    i1.wait()
    i2.wait()
    gh = [None] * _QCH
    wh = [None] * _QCH
    for j in range(_PA):
        gh[j] = (pltpu.async_copy(u_hbm.at[qs_v.at[j]], bu_v.at[j % _PD],
                                  gsems[j % _PD]),
                 pltpu.async_copy(v_hbm.at[qt_v.at[j]], bv_v.at[j % _PD],
                                  gsems[j % _PD]))
    for j in range(_QCH):
        b = j % _PD
        gh[j][0].wait()
        gh[j][1].wait()
        off = w * _PAIRS_PER_TILE + j * _PCH
        wh[j] = (pltpu.async_copy(bu_v.at[b], out_hbm.at[pl.ds(off, _PCH)],
                                  wsems[b]),
                 pltpu.async_copy(bv_v.at[b],
                                  out_hbm.at[pl.ds(T * QP + off, _PCH)],
                                  wsems[b]))
        nx = j + _PA
        if nx < _QCH:
            old = nx - _PD
            if old >= 0:
                wh[old][0].wait()
                wh[old][1].wait()
            gh[nx] = (pltpu.async_copy(u_hbm.at[qs_v.at[nx]],
                                       bu_v.at[nx % _PD], gsems[nx % _PD]),
                      pltpu.async_copy(v_hbm.at[qt_v.at[nx]],
                                       bv_v.at[nx % _PD], gsems[nx % _PD]))
    for j in range(_QCH - _PD, _QCH):
        if wh[j] is not None:
            wh[j][0].wait()
            wh[j][1].wait()


def kernel(x, edge_index, query_src, query_tgt, W_in, b_in, W_msg, W_self, b_h,
           type_emb, W_dec1, b_dec1, w_dec2, b_dec2):
    x = x.astype(f32)
    xp = jnp.zeros((NP, FP), f32).at[:N, :F].set(x)
    wip = jnp.zeros((FP, H), f32).at[:F, :].set(W_in.astype(f32))

    src = edge_index[:, 0, :].astype(jnp.int32)
    dst = edge_index[:, 1, :].astype(jnp.int32)
    pad_src = jnp.broadcast_to(jnp.arange(EP - E, dtype=jnp.int32) % N,
                               (T, EP - E))
    pad_dst = jnp.broadcast_to(
        N + (jnp.arange(EP - E, dtype=jnp.int32) % (NP - N)), (T, EP - E))
    srcp = jnp.concatenate(
        [src, pad_src], axis=1).reshape(T * NCORE * NTILE, _NCH, CH)
    dstp = jnp.concatenate(
        [dst, pad_dst], axis=1).reshape(T * NCORE * NTILE, _NCH, CH)
    qs = jnp.concatenate(
        [query_src.astype(jnp.int32), jnp.zeros((T, QP - Q), jnp.int32)],
        axis=1).reshape(T * QP).reshape(NCORE * NTILE, _QCH, _PCH)
    qt = jnp.concatenate(
        [query_tgt.astype(jnp.int32), jnp.zeros((T, QP - Q), jnp.int32)],
        axis=1).reshape(T * QP).reshape(NCORE * NTILE, _QCH, _PCH)
    zeros_tile = jnp.zeros((ZB, H), f32)
    ones_chunk = jnp.ones((CH, H), f32)
    wd = W_dec1.astype(f32)

    h = _inproj(xp, wip, b_in.reshape(1, H).astype(f32))
    sall = _seg0(h, srcp, dstp, zeros_tile, ones_chunk)
    s4 = sall[:T * NCORE * NP].reshape(T, NCORE, NP, H)
    cntacc = sall[T * NCORE * NP:].reshape(T, NCORE, NP, H)
    recip = _cntprep(cntacc)
    h = _layer(h, s4, recip, W_self[0].astype(f32), W_msg[0].astype(f32),
               b_h[0].reshape(1, H).astype(f32))
    for l in range(1, L):
        s_flat = _seg(h, srcp, dstp, zeros_tile)
        s4 = s_flat.reshape(T, NCORE, NP, H)
        if l < L - 1:
            h = _layer(h, s4, recip, W_self[l].astype(f32),
                       W_msg[l].astype(f32),
                       b_h[l].reshape(1, H).astype(f32))
        else:
            u, v = _layer_uv(h, s4, recip, W_self[l].astype(f32),
                             W_msg[l].astype(f32),
                             b_h[l].reshape(1, H).astype(f32),
                             wd[:H], wd[H:2 * H])
    g = _pair(u, v, qs, qt)
    g3 = g.reshape(2, T * QP, H)
    logits = _dec(g3, g3, type_emb.astype(f32), wd[2 * H:],
                  b_dec1.reshape(1, H).astype(f32),
                  w_dec2.reshape(1, H).astype(f32),
                  jnp.reshape(b_dec2, (1,)).astype(f32))
    return logits.reshape(T, QP)[:, :Q]


# speedup vs baseline: 1.0976x; 1.0976x over previous
import functools

import jax
import jax.numpy as jnp
from jax import lax
from jax.experimental import pallas as pl
from jax.experimental.pallas import tpu as pltpu
from jax.experimental.pallas import tpu_sc as plsc

N = 10000
E = 32000
T = 5
F = 404
H = 128
L = 3
Q = 32000

NP = 10240
FP = 512
EP = 32768
QP = 32768
CH = 128
ZB = 64
RB = 512
QB = 512
NTILE = 16
NCORE = 2

f32 = jnp.float32


def _r16(v):
    return v.astype(jnp.bfloat16).astype(f32)


def _inproj_body(x_ref, w_ref, b_ref, o_ref):
    o_ref[...] = _r16(jnp.dot(_r16(x_ref[...]), _r16(w_ref[...]),
                              preferred_element_type=f32,
                              precision=lax.Precision.HIGHEST) + b_ref[...])


_inproj = pl.pallas_call(
    _inproj_body,
    grid=(NP // RB,),
    in_specs=[
        pl.BlockSpec((RB, FP), lambda i: (i, 0)),
        pl.BlockSpec((FP, H), lambda i: (0, 0)),
        pl.BlockSpec((1, H), lambda i: (0, 0)),
    ],
    out_specs=pl.BlockSpec((RB, H), lambda i: (i, 0)),
    out_shape=jax.ShapeDtypeStruct((NP, H), f32),
)


def _cntprep_body(c_ref, o_ref):
    cnt = c_ref[0, 0, :, 0:1] + c_ref[0, 1, :, 0:1]
    recip = 1.0 / jnp.clip(cnt, 1.0, None)
    o_ref[...] = jnp.broadcast_to(recip, (RB, 8))[None]


_cntprep = pl.pallas_call(
    _cntprep_body,
    grid=(T, NP // RB),
    in_specs=[pl.BlockSpec((1, NCORE, RB, H), lambda t, i: (t, 0, i, 0))],
    out_specs=pl.BlockSpec((1, RB, 8), lambda t, i: (t, i, 0)),
    out_shape=jax.ShapeDtypeStruct((T, NP, 8), f32),
)


def _layer_body(h_ref, s_ref, r_ref, wself_ref, wmsg_ref, b_ref, o_ref):
    acc = jnp.dot(h_ref[...], _r16(wself_ref[...]),
                  preferred_element_type=f32, precision=lax.Precision.HIGHEST)
    for t in range(T):
        st = s_ref[t, 0] + s_ref[t, 1]
        recip = r_ref[t, :, 0:1]
        acc = acc + jnp.dot(st, _r16(wmsg_ref[t]),
                            preferred_element_type=f32,
                            precision=lax.Precision.HIGHEST) * recip
    o_ref[...] = _r16(jnp.maximum(acc + b_ref[...], 0.0))


_layer = pl.pallas_call(
    _layer_body,
    grid=(NP // RB,),
    in_specs=[
        pl.BlockSpec((RB, H), lambda i: (i, 0)),
        pl.BlockSpec((T, NCORE, RB, H), lambda i: (0, 0, i, 0)),
        pl.BlockSpec((T, RB, 8), lambda i: (0, i, 0)),
        pl.BlockSpec((H, H), lambda i: (0, 0)),
        pl.BlockSpec((T, H, H), lambda i: (0, 0, 0)),
        pl.BlockSpec((1, H), lambda i: (0, 0)),
    ],
    out_specs=pl.BlockSpec((RB, H), lambda i: (i, 0)),
    out_shape=jax.ShapeDtypeStruct((NP, H), f32),
)


def _layer_uv_body(h_ref, s_ref, r_ref, wself_ref, wmsg_ref, b_ref,
                   wa_ref, wb_ref, u_ref, v_ref):
    acc = jnp.dot(h_ref[...], _r16(wself_ref[...]),
                  preferred_element_type=f32, precision=lax.Precision.HIGHEST)
    for t in range(T):
        st = s_ref[t, 0] + s_ref[t, 1]
        recip = r_ref[t, :, 0:1]
        acc = acc + jnp.dot(st, _r16(wmsg_ref[t]),
                            preferred_element_type=f32,
                            precision=lax.Precision.HIGHEST) * recip
    out = _r16(jnp.maximum(acc + b_ref[...], 0.0))
    u_ref[...] = jnp.dot(out, _r16(wa_ref[...]),
                         preferred_element_type=f32,
                         precision=lax.Precision.HIGHEST)
    v_ref[...] = jnp.dot(out, _r16(wb_ref[...]),
                         preferred_element_type=f32,
                         precision=lax.Precision.HIGHEST)


_layer_uv = pl.pallas_call(
    _layer_uv_body,
    grid=(NP // RB,),
    in_specs=[
        pl.BlockSpec((RB, H), lambda i: (i, 0)),
        pl.BlockSpec((T, NCORE, RB, H), lambda i: (0, 0, i, 0)),
        pl.BlockSpec((T, RB, 8), lambda i: (0, i, 0)),
        pl.BlockSpec((H, H), lambda i: (0, 0)),
        pl.BlockSpec((T, H, H), lambda i: (0, 0, 0)),
        pl.BlockSpec((1, H), lambda i: (0, 0)),
        pl.BlockSpec((H, H), lambda i: (0, 0)),
        pl.BlockSpec((H, H), lambda i: (0, 0)),
    ],
    out_specs=[
        pl.BlockSpec((RB, H), lambda i: (i, 0)),
        pl.BlockSpec((RB, H), lambda i: (i, 0)),
    ],
    out_shape=[
        jax.ShapeDtypeStruct((NP, H), f32),
        jax.ShapeDtypeStruct((NP, H), f32),
    ],
)


_QBLOCKS_PER_T = QP // QB


def _dec_body(gu_ref, gv_ref, temb_ref, w1c_ref, b1_ref, w2_ref, b2_ref, o_ref):
    t = pl.program_id(0) // _QBLOCKS_PER_T
    et = temb_ref[pl.ds(t, 1), :]
    c = jnp.dot(_r16(et), _r16(w1c_ref[...]),
                preferred_element_type=f32,
                precision=lax.Precision.HIGHEST) + b1_ref[...]
    z = _r16(jnp.maximum(gu_ref[0] + gv_ref[0] + c, 0.0))
    o_ref[...] = jnp.sum(z * _r16(w2_ref[...]), axis=1) + b2_ref[0]


_dec = pl.pallas_call(
    _dec_body,
    grid=(T * QP // QB,),
    in_specs=[
        pl.BlockSpec((1, QB, H), lambda i: (0, i, 0)),
        pl.BlockSpec((1, QB, H), lambda i: (1, i, 0)),
        pl.BlockSpec((T, H), lambda i: (0, 0)),
        pl.BlockSpec((H, H), lambda i: (0, 0)),
        pl.BlockSpec((1, H), lambda i: (0, 0)),
        pl.BlockSpec((1, H), lambda i: (0, 0)),
        pl.BlockSpec(memory_space=pltpu.SMEM),
    ],
    out_specs=pl.BlockSpec((QB,), lambda i: (i,)),
    out_shape=jax.ShapeDtypeStruct((T * QP,), f32),
)


_sc_mesh = plsc.VectorSubcoreMesh(core_axis_name="c", subcore_axis_name="s")

_ETILE = EP // (NCORE * NTILE)
_NCH = _ETILE // CH
_STRIPE = NP // NTILE


@functools.partial(
    pl.kernel,
    mesh=_sc_mesh,
    out_type=jax.ShapeDtypeStruct((T * NCORE * NP, H), f32),
    scratch_types=[
        pltpu.VMEM((_NCH, CH), jnp.int32),
        pltpu.VMEM((_NCH, CH), jnp.int32),
        pltpu.VMEM((2, CH, H), f32),
        pltpu.VMEM((ZB, H), f32),
        pltpu.VMEM_SHARED((NP, H), f32),
        pltpu.SemaphoreType.DMA,
        pltpu.SemaphoreType.DMA,
    ],
)
def _seg(h_hbm, src_hbm, dst_hbm, zeros_hbm, out_hbm,
         src_v, dst_v, rows_v, zero_v, acc_sh, sem0, sem1):
    c = lax.axis_index("c")
    s = lax.axis_index("s")
    pltpu.sync_copy(zeros_hbm, zero_v)
    for t in range(T):
        r = (t * NCORE + c) * NTILE + s
        pltpu.sync_copy(src_hbm.at[r], src_v)
        pltpu.sync_copy(dst_hbm.at[r], dst_v)
        for z in range(_STRIPE // ZB):
            pltpu.sync_copy(zero_v, acc_sh.at[pl.ds(s * _STRIPE + z * ZB, ZB)])
        plsc.subcore_barrier()
        sems = [sem0, sem1]
        gh = [None] * _NCH
        gh[0] = pltpu.async_copy(h_hbm.at[src_v.at[0]], rows_v.at[0], sem0)
        for j in range(_NCH):
            b = j % 2
            gh[j].wait()
            if j + 1 < _NCH:
                gh[j + 1] = pltpu.async_copy(h_hbm.at[src_v.at[j + 1]],
                                             rows_v.at[1 - b], sems[1 - b])
            pltpu.sync_copy(rows_v.at[b], acc_sh.at[dst_v.at[j]], add=True)
        plsc.subcore_barrier()
        out_row = (t * NCORE + c) * NP + s * _STRIPE
        pltpu.sync_copy(acc_sh.at[pl.ds(s * _STRIPE, _STRIPE)],
                        out_hbm.at[pl.ds(out_row, _STRIPE)])


_PAIRS_PER_TILE = T * QP // (NCORE * NTILE)
_PCH = 64
_QCH = _PAIRS_PER_TILE // _PCH
_PD = 6
_PA = 3


@functools.partial(
    pl.kernel,
    mesh=_sc_mesh,
    out_type=jax.ShapeDtypeStruct((2 * T * QP, H), f32),
    scratch_types=[
        pltpu.VMEM((_QCH, _PCH), jnp.int32),
        pltpu.VMEM((_QCH, _PCH), jnp.int32),
        pltpu.VMEM((_PD, _PCH, H), f32),
        pltpu.VMEM((_PD, _PCH, H), f32),
        pltpu.SemaphoreType.DMA,
        pltpu.SemaphoreType.DMA,
        pltpu.SemaphoreType.DMA,
        pltpu.SemaphoreType.DMA,
        pltpu.SemaphoreType.DMA,
        pltpu.SemaphoreType.DMA,
        pltpu.SemaphoreType.DMA,
        pltpu.SemaphoreType.DMA,
        pltpu.SemaphoreType.DMA,
        pltpu.SemaphoreType.DMA,
        pltpu.SemaphoreType.DMA,
        pltpu.SemaphoreType.DMA,
        pltpu.SemaphoreType.DMA,
    ],
)
def _pair(u_hbm, v_hbm, qs_hbm, qt_hbm, out_hbm,
          qs_v, qt_v, bu_v, bv_v, isem,
          gsem0, gsem1, gsem2, gsem3, gsem4, gsem5,
          wsem0, wsem1, wsem2, wsem3, wsem4, wsem5):
    c = lax.axis_index("c")
    s = lax.axis_index("s")
    w = c * NTILE + s
    gsems = [gsem0, gsem1, gsem2, gsem3, gsem4, gsem5]
    wsems = [wsem0, wsem1, wsem2, wsem3, wsem4, wsem5]
    i1 = pltpu.async_copy(qs_hbm.at[w], qs_v, isem)
    i2 = pltpu.async_copy(qt_hbm.at[w], qt_v, isem)
    i1.wait()
    i2.wait()
    gh = [None] * _QCH
    wh = [None] * _QCH
    for j in range(_PA):
        gh[j] = (pltpu.async_copy(u_hbm.at[qs_v.at[j]], bu_v.at[j % _PD],
                                  gsems[j % _PD]),
                 pltpu.async_copy(v_hbm.at[qt_v.at[j]], bv_v.at[j % _PD],
                                  gsems[j % _PD]))
    for j in range(_QCH):
        b = j % _PD
        gh[j][0].wait()
        gh[j][1].wait()
        off = w * _PAIRS_PER_TILE + j * _PCH
        wh[j] = (pltpu.async_copy(bu_v.at[b], out_hbm.at[pl.ds(off, _PCH)],
                                  wsems[b]),
                 pltpu.async_copy(bv_v.at[b],
                                  out_hbm.at[pl.ds(T * QP + off, _PCH)],
                                  wsems[b]))
        nx = j + _PA
        if nx < _QCH:
            old = nx - _PD
            if old >= 0:
                wh[old][0].wait()
                wh[old][1].wait()
            gh[nx] = (pltpu.async_copy(u_hbm.at[qs_v.at[nx]],
                                       bu_v.at[nx % _PD], gsems[nx % _PD]),
                      pltpu.async_copy(v_hbm.at[qt_v.at[nx]],
                                       bv_v.at[nx % _PD], gsems[nx % _PD]))
    for j in range(_QCH - _PD, _QCH):
        if wh[j] is not None:
            wh[j][0].wait()
            wh[j][1].wait()


def kernel(x, edge_index, query_src, query_tgt, W_in, b_in, W_msg, W_self, b_h,
           type_emb, W_dec1, b_dec1, w_dec2, b_dec2):
    x = x.astype(f32)
    xp = jnp.zeros((NP, FP), f32).at[:N, :F].set(x)
    wip = jnp.zeros((FP, H), f32).at[:F, :].set(W_in.astype(f32))

    src = edge_index[:, 0, :].astype(jnp.int32)
    dst = edge_index[:, 1, :].astype(jnp.int32)
    pad_src = jnp.broadcast_to(jnp.arange(EP - E, dtype=jnp.int32) % N,
                               (T, EP - E))
    pad_dst = jnp.broadcast_to(
        N + (jnp.arange(EP - E, dtype=jnp.int32) % (NP - N)), (T, EP - E))
    srcp = jnp.concatenate(
        [src, pad_src], axis=1).reshape(T * NCORE * NTILE, _NCH, CH)
    dstp = jnp.concatenate(
        [dst, pad_dst], axis=1).reshape(T * NCORE * NTILE, _NCH, CH)
    qs = jnp.concatenate(
        [query_src.astype(jnp.int32), jnp.zeros((T, QP - Q), jnp.int32)],
        axis=1).reshape(T * QP).reshape(NCORE * NTILE, _QCH, _PCH)
    qt = jnp.concatenate(
        [query_tgt.astype(jnp.int32), jnp.zeros((T, QP - Q), jnp.int32)],
        axis=1).reshape(T * QP).reshape(NCORE * NTILE, _QCH, _PCH)
    zeros_tile = jnp.zeros((ZB, H), f32)
    ones_table = jnp.ones((NP, H), f32)
    wd = W_dec1.astype(f32)

    cntacc = _seg(ones_table, srcp, dstp, zeros_tile)
    recip = _cntprep(cntacc.reshape(T, NCORE, NP, H))

    h = _inproj(xp, wip, b_in.reshape(1, H).astype(f32))
    for l in range(L):
        s_flat = _seg(h, srcp, dstp, zeros_tile)
        s4 = s_flat.reshape(T, NCORE, NP, H)
        if l < L - 1:
            h = _layer(h, s4, recip, W_self[l].astype(f32),
                       W_msg[l].astype(f32),
                       b_h[l].reshape(1, H).astype(f32))
        else:
            u, v = _layer_uv(h, s4, recip, W_self[l].astype(f32),
                             W_msg[l].astype(f32),
                             b_h[l].reshape(1, H).astype(f32),
                             wd[:H], wd[H:2 * H])
    g = _pair(u, v, qs, qt)
    g3 = g.reshape(2, T * QP, H)
    logits = _dec(g3, g3, type_emb.astype(f32), wd[2 * H:],
                  b_dec1.reshape(1, H).astype(f32),
                  w_dec2.reshape(1, H).astype(f32),
                  jnp.reshape(b_dec2, (1,)).astype(f32))
    return logits.reshape(T, QP)[:, :Q]
